# baseline (speedup 1.0000x reference)
import functools

import jax
import jax.numpy as jnp
from jax import lax
from jax.experimental import pallas as pl
from jax.experimental.pallas import tpu as pltpu
from jax.experimental.pallas import tpu_sc as plsc

N_NODES = 10000
N_EDGES = 320000
D = 128
NC = 2
NS = 16
NW = NC * NS
N_PAD = 10240
E_PAD = 327680
CH = 128
EPW0 = 16384
EPW1 = 4096
SEG = EPW0 + EPW1
CPW0 = EPW0 // CH
CPW1 = EPW1 // CH
RPT = N_PAD // NS
CH2 = 1024
CPW20 = EPW0 // CH2
CPW21 = EPW1 // CH2

_F32 = jnp.float32


def _mesh():
    return plsc.VectorSubcoreMesh(
        core_axis_name="c", subcore_axis_name="s",
        num_cores=NC, num_subcores=NS)


@functools.partial(
    pl.kernel,
    out_type=jax.ShapeDtypeStruct((NW, N_PAD), _F32),
    mesh=_mesh(),
    scratch_types=[
        pltpu.VMEM((CH2,), jnp.int32),
        pltpu.VMEM((CH2,), jnp.int32),
        pltpu.VMEM((N_PAD,), _F32),
        pltpu.SemaphoreType.DMA,
        pltpu.SemaphoreType.DMA,
    ],
    compiler_params=pltpu.CompilerParams(needs_layout_passes=False),
)
def _deg_kernel(src_hbm, out_hbm, idx0, idx1, cs, sem0, sem1):
    cid = lax.axis_index("c")
    sid = lax.axis_index("s")
    wid = sid * NC + cid
    ebase = sid * SEG + cid * EPW0
    ncpw2 = jnp.where(cid == 0, CPW20, CPW21)
    pltpu.async_copy(src_hbm.at[pl.ds(ebase, CH2)], idx0, sem0)
    z = jnp.zeros((16,), _F32)

    def zbody(i, carry):
        cs[pl.ds(i * 64, 16)] = z
        cs[pl.ds(i * 64 + 16, 16)] = z
        cs[pl.ds(i * 64 + 32, 16)] = z
        cs[pl.ds(i * 64 + 48, 16)] = z
        return carry

    lax.fori_loop(0, N_PAD // 64, zbody, 0)
    ones16 = jnp.ones((16,), _F32)

    def count(buf):
        for j in range(CH2 // 16):
            plsc.addupdate_scatter(cs, [buf[pl.ds(j * 16, 16)]], ones16)

    def body(i, carry):
        k0 = 2 * i
        pltpu.make_async_copy(
            src_hbm.at[pl.ds(ebase + k0 * CH2, CH2)], idx0, sem0).wait()
        pltpu.async_copy(
            src_hbm.at[pl.ds(ebase + ((k0 + 1) % ncpw2) * CH2, CH2)], idx1, sem1)
        count(idx0)
        pltpu.make_async_copy(
            src_hbm.at[pl.ds(ebase + ((k0 + 1) % ncpw2) * CH2, CH2)], idx1, sem1).wait()
        pltpu.async_copy(
            src_hbm.at[pl.ds(ebase + ((k0 + 2) % ncpw2) * CH2, CH2)], idx0, sem0)
        count(idx1)
        return carry

    lax.fori_loop(0, ncpw2 // 2, body, 0)
    pltpu.make_async_copy(src_hbm.at[pl.ds(ebase, CH2)], idx0, sem0).wait()
    pltpu.sync_copy(cs, out_hbm.at[wid])


@functools.partial(
    pl.kernel,
    out_type=(jax.ShapeDtypeStruct((NC, N_PAD, D), _F32),
              jax.ShapeDtypeStruct((NW, N_PAD), _F32)),
    mesh=_mesh(),
    scratch_types=[
        pltpu.VMEM((CH,), jnp.int32),
        pltpu.VMEM((CH,), jnp.int32),
        pltpu.VMEM((CH,), jnp.int32),
        pltpu.VMEM((CH,), jnp.int32),
        pltpu.VMEM((CH, D), _F32),
        pltpu.VMEM((CH, D), _F32),
        pltpu.VMEM((N_PAD,), _F32),
        pltpu.VMEM_SHARED((N_PAD, D), _F32),
        pltpu.SemaphoreType.DMA,
        pltpu.SemaphoreType.DMA,
        pltpu.SemaphoreType.DMA,
        pltpu.SemaphoreType.DMA,
        pltpu.SemaphoreType.DMA,
        pltpu.SemaphoreType.DMA,
    ],
    compiler_params=pltpu.CompilerParams(needs_layout_passes=False),
)
def _agg_kernel(h_hbm, src_hbm, dst_hbm, zrows_hbm, out_hbm, cd_hbm,
                s0, s1, d0, d1, r0, r1, cd, acc_sh,
                g0, g1, q0, q1, g2, g3):
    cid = lax.axis_index("c")
    sid = lax.axis_index("s")
    wid = sid * NC + cid
    ebase = sid * SEG + cid * EPW0
    ncpw = jnp.where(cid == 0, CPW0, CPW1)
    sbufs = (s0, s1)
    dbufs = (d0, d1)
    rbufs = (r0, r1)
    gsems = (g0, g1)
    hsems = (g2, g3)
    HCH = CH // 2
    qsems = (q0, q1)

    pltpu.sync_copy(src_hbm.at[pl.ds(ebase, CH)], s0)
    pltpu.async_copy(h_hbm.at[s0.at[pl.ds(0, HCH)]], r0.at[pl.ds(0, HCH)], g0)
    pltpu.async_copy(h_hbm.at[s0.at[pl.ds(HCH, HCH)]], r0.at[pl.ds(HCH, HCH)], g2)
    pltpu.sync_copy(src_hbm.at[pl.ds(ebase + CH, CH)], s1)
    pltpu.async_copy(h_hbm.at[s1.at[pl.ds(0, HCH)]], r1.at[pl.ds(0, HCH)], g1)
    pltpu.async_copy(h_hbm.at[s1.at[pl.ds(HCH, HCH)]], r1.at[pl.ds(HCH, HCH)], g3)

    z = jnp.zeros((16,), _F32)

    def zbody(i, carry):
        cd[pl.ds(i * 64, 16)] = z
        cd[pl.ds(i * 64 + 16, 16)] = z
        cd[pl.ds(i * 64 + 32, 16)] = z
        cd[pl.ds(i * 64 + 48, 16)] = z
        return carry

    lax.fori_loop(0, N_PAD // 64, zbody, 0)
    pltpu.sync_copy(zrows_hbm, acc_sh.at[pl.ds(sid * RPT, RPT)])
    plsc.subcore_barrier()

    ones16 = jnp.ones((16,), _F32)

    def consume(k, b):
        pltpu.make_async_copy(h_hbm.at[sbufs[b].at[pl.ds(0, HCH)]],
                              rbufs[b].at[pl.ds(0, HCH)], gsems[b]).wait()
        pltpu.make_async_copy(h_hbm.at[sbufs[b].at[pl.ds(HCH, HCH)]],
                              rbufs[b].at[pl.ds(HCH, HCH)], hsems[b]).wait()
        pltpu.sync_copy(dst_hbm.at[pl.ds(ebase + k * CH, CH)], dbufs[b])
        for j in range(CH // 16):
            plsc.addupdate_scatter(cd, [dbufs[b][pl.ds(j * 16, 16)]], ones16)
        pltpu.async_copy(rbufs[b], acc_sh.at[dbufs[b]], qsems[b], add=True)

    def prefetch(k2, b2):
        pltpu.sync_copy(
            src_hbm.at[pl.ds(ebase + (k2 % ncpw) * CH, CH)], sbufs[b2])
        pltpu.make_async_copy(rbufs[b2], acc_sh.at[dbufs[b2]],
                              qsems[b2]).wait()
        pltpu.async_copy(h_hbm.at[sbufs[b2].at[pl.ds(0, HCH)]],
                         rbufs[b2].at[pl.ds(0, HCH)], gsems[b2])
        pltpu.async_copy(h_hbm.at[sbufs[b2].at[pl.ds(HCH, HCH)]],
                         rbufs[b2].at[pl.ds(HCH, HCH)], hsems[b2])

    consume(0, 0)
    prefetch(2, 0)
    consume(1, 1)
    prefetch(3, 1)

    def body(i, carry):
        k = 2 * i + 2
        consume(k, 0)
        prefetch(k + 2, 0)
        consume(k + 1, 1)
        prefetch(k + 3, 1)
        return carry

    lax.fori_loop(0, (ncpw - 2) // 2, body, 0)
    pltpu.make_async_copy(h_hbm.at[s0.at[pl.ds(0, HCH)]], r0.at[pl.ds(0, HCH)], g0).wait()
    pltpu.make_async_copy(h_hbm.at[s0.at[pl.ds(HCH, HCH)]], r0.at[pl.ds(HCH, HCH)], g2).wait()
    pltpu.make_async_copy(h_hbm.at[s1.at[pl.ds(0, HCH)]], r1.at[pl.ds(0, HCH)], g1).wait()
    pltpu.make_async_copy(h_hbm.at[s1.at[pl.ds(HCH, HCH)]], r1.at[pl.ds(HCH, HCH)], g3).wait()
    plsc.subcore_barrier()
    pltpu.sync_copy(acc_sh.at[pl.ds(sid * RPT, RPT)],
                    out_hbm.at[cid, pl.ds(sid * RPT, RPT)])
    pltpu.sync_copy(cd, cd_hbm.at[wid])


def _scale_body(cnt_ref, feat_ref, h_ref):
    deg = jnp.maximum(jnp.sum(cnt_ref[...], axis=1, keepdims=True), 1.0)
    h_ref[...] = feat_ref[...] * lax.rsqrt(deg)


_scale = pl.pallas_call(
    _scale_body,
    grid=(N_PAD // 1024,),
    in_specs=[pl.BlockSpec((1024, NW), lambda i: (i, 0)),
              pl.BlockSpec((1024, D), lambda i: (i, 0))],
    out_specs=pl.BlockSpec((1024, D), lambda i: (i, 0)),
    out_shape=jax.ShapeDtypeStruct((N_PAD, D), _F32),
)


def _final_body(a0_ref, a1_ref, cnt_ref, w_ref, b_ref, o_ref):
    a = a0_ref[...] + a1_ref[...]
    r = jnp.dot(a, w_ref[...], preferred_element_type=_F32)
    nd = lax.rsqrt(jnp.maximum(jnp.sum(cnt_ref[...], axis=1, keepdims=True), 1.0))
    o_ref[...] = r * nd + b_ref[...]


_final = pl.pallas_call(
    _final_body,
    grid=(N_PAD // 1024,),
    in_specs=[pl.BlockSpec((1024, D), lambda i: (i, 0)),
              pl.BlockSpec((1024, D), lambda i: (i, 0)),
              pl.BlockSpec((1024, NW), lambda i: (i, 0)),
              pl.BlockSpec((D, D), lambda i: (0, 0)),
              pl.BlockSpec((1, D), lambda i: (0, 0))],
    out_specs=pl.BlockSpec((1024, D), lambda i: (i, 0)),
    out_shape=jax.ShapeDtypeStruct((N_PAD, D), _F32),
)


def kernel(feat, edge_index, W, b):
    src = edge_index[0]
    dst = edge_index[1]
    pad = jnp.full((E_PAD - N_EDGES,), N_NODES, jnp.int32)
    src_p = jnp.concatenate([src, pad])
    dst_p = jnp.concatenate([dst, pad])
    feat_p = jnp.pad(feat, ((0, N_PAD - N_NODES), (0, 0)))

    cnt_s = _deg_kernel(src_p).T
    h = _scale(cnt_s, feat_p)

    zrows = jnp.zeros((RPT, D), _F32)
    acc, cd_part = _agg_kernel(h, src_p, dst_p, zrows)

    out = _final(acc[0], acc[1], cd_part.T, W, b.reshape(1, D))
    return out[:N_NODES]


# speedup vs baseline: 1.0406x; 1.0406x over previous
import functools

import jax
import jax.numpy as jnp
from jax import lax
from jax.experimental import pallas as pl
from jax.experimental.pallas import tpu as pltpu
from jax.experimental.pallas import tpu_sc as plsc

N_NODES = 10000
N_EDGES = 320000
D = 128
NC = 2
NS = 16
NW = NC * NS
N_PAD = 10240
E_PAD = 327680
CH = 128
EPW0 = 15360
EPW1 = 5120
SEG = EPW0 + EPW1
CPW0 = EPW0 // CH
CPW1 = EPW1 // CH
RPT = N_PAD // NS
CH2 = 512
CPW20 = EPW0 // CH2
CPW21 = EPW1 // CH2

_F32 = jnp.float32


def _mesh():
    return plsc.VectorSubcoreMesh(
        core_axis_name="c", subcore_axis_name="s",
        num_cores=NC, num_subcores=NS)


@functools.partial(
    pl.kernel,
    out_type=jax.ShapeDtypeStruct((NW, N_PAD), _F32),
    mesh=_mesh(),
    scratch_types=[
        pltpu.VMEM((CH2,), jnp.int32),
        pltpu.VMEM((CH2,), jnp.int32),
        pltpu.VMEM((N_PAD,), _F32),
        pltpu.SemaphoreType.DMA,
        pltpu.SemaphoreType.DMA,
    ],
    compiler_params=pltpu.CompilerParams(needs_layout_passes=False),
)
def _deg_kernel(src_hbm, out_hbm, idx0, idx1, cs, sem0, sem1):
    cid = lax.axis_index("c")
    sid = lax.axis_index("s")
    wid = sid * NC + cid
    ebase = sid * SEG + cid * EPW0
    pltpu.async_copy(src_hbm.at[pl.ds(ebase, CH2)], idx0, sem0)
    z = jnp.zeros((16,), _F32)

    def zbody(i, carry):
        cs[pl.ds(i * 64, 16)] = z
        cs[pl.ds(i * 64 + 16, 16)] = z
        cs[pl.ds(i * 64 + 32, 16)] = z
        cs[pl.ds(i * 64 + 48, 16)] = z
        return carry

    lax.fori_loop(0, N_PAD // 64, zbody, 0)
    ones16 = jnp.ones((16,), _F32)

    def count(buf):
        for j in range(CH2 // 16):
            plsc.addupdate_scatter(cs, [buf[pl.ds(j * 16, 16)]], ones16)

    def run(ncpw2):
        def body(i, carry):
            k0 = 2 * i
            pltpu.make_async_copy(
                src_hbm.at[pl.ds(ebase + k0 * CH2, CH2)], idx0, sem0).wait()
            pltpu.async_copy(
                src_hbm.at[pl.ds(ebase + ((k0 + 1) % ncpw2) * CH2, CH2)], idx1, sem1)
            count(idx0)
            pltpu.make_async_copy(
                src_hbm.at[pl.ds(ebase + ((k0 + 1) % ncpw2) * CH2, CH2)], idx1, sem1).wait()
            pltpu.async_copy(
                src_hbm.at[pl.ds(ebase + ((k0 + 2) % ncpw2) * CH2, CH2)], idx0, sem0)
            count(idx1)
            return carry

        lax.fori_loop(0, ncpw2 // 2, body, 0)

    pl.when(cid == 0)(lambda: run(CPW20))
    pl.when(cid != 0)(lambda: run(CPW21))
    pltpu.make_async_copy(src_hbm.at[pl.ds(ebase, CH2)], idx0, sem0).wait()
    pltpu.sync_copy(cs, out_hbm.at[wid])


@functools.partial(
    pl.kernel,
    out_type=(jax.ShapeDtypeStruct((NC, N_PAD, D), _F32),
              jax.ShapeDtypeStruct((NW, N_PAD), _F32)),
    mesh=_mesh(),
    scratch_types=[
        pltpu.VMEM((CH,), jnp.int32),
        pltpu.VMEM((CH,), jnp.int32),
        pltpu.VMEM((CH,), jnp.int32),
        pltpu.VMEM((CH,), jnp.int32),
        pltpu.VMEM((CH, D), _F32),
        pltpu.VMEM((CH, D), _F32),
        pltpu.VMEM((N_PAD,), _F32),
        pltpu.VMEM_SHARED((N_PAD, D), _F32),
        pltpu.SemaphoreType.DMA,
        pltpu.SemaphoreType.DMA,
        pltpu.SemaphoreType.DMA,
        pltpu.SemaphoreType.DMA,
    ],
    compiler_params=pltpu.CompilerParams(needs_layout_passes=False),
)
def _agg_kernel(h_hbm, src_hbm, dst_hbm, zrows_hbm, out_hbm, cd_hbm,
                s0, s1, d0, d1, r0, r1, cd, acc_sh,
                g0, g1, q0, q1):
    cid = lax.axis_index("c")
    sid = lax.axis_index("s")
    wid = sid * NC + cid
    ebase = sid * SEG + cid * EPW0
    sbufs = (s0, s1)
    dbufs = (d0, d1)
    rbufs = (r0, r1)
    gsems = (g0, g1)
    qsems = (q0, q1)

    pltpu.sync_copy(src_hbm.at[pl.ds(ebase, CH)], s0)
    pltpu.async_copy(h_hbm.at[s0], r0, g0)
    pltpu.sync_copy(src_hbm.at[pl.ds(ebase + CH, CH)], s1)
    pltpu.async_copy(h_hbm.at[s1], r1, g1)

    z = jnp.zeros((16,), _F32)

    def zbody(i, carry):
        cd[pl.ds(i * 64, 16)] = z
        cd[pl.ds(i * 64 + 16, 16)] = z
        cd[pl.ds(i * 64 + 32, 16)] = z
        cd[pl.ds(i * 64 + 48, 16)] = z
        return carry

    lax.fori_loop(0, N_PAD // 64, zbody, 0)
    pltpu.sync_copy(zrows_hbm, acc_sh.at[pl.ds(sid * RPT, RPT)])
    plsc.subcore_barrier()

    ones16 = jnp.ones((16,), _F32)

    def consume(k, b):
        pltpu.make_async_copy(h_hbm.at[sbufs[b]], rbufs[b], gsems[b]).wait()
        pltpu.sync_copy(dst_hbm.at[pl.ds(ebase + k * CH, CH)], dbufs[b])
        for j in range(CH // 16):
            plsc.addupdate_scatter(cd, [dbufs[b][pl.ds(j * 16, 16)]], ones16)
        pltpu.async_copy(rbufs[b], acc_sh.at[dbufs[b]], qsems[b], add=True)

    def run(ncpw):
        def prefetch(k2, b2):
            pltpu.sync_copy(
                src_hbm.at[pl.ds(ebase + (k2 % ncpw) * CH, CH)], sbufs[b2])
            pltpu.make_async_copy(rbufs[b2], acc_sh.at[dbufs[b2]],
                                  qsems[b2]).wait()
            pltpu.async_copy(h_hbm.at[sbufs[b2]], rbufs[b2], gsems[b2])

        consume(0, 0)
        prefetch(2, 0)
        consume(1, 1)
        prefetch(3, 1)

        def body(i, carry):
            k = 2 * i + 2
            consume(k, 0)
            prefetch(k + 2, 0)
            consume(k + 1, 1)
            prefetch(k + 3, 1)
            return carry

        lax.fori_loop(0, (ncpw - 2) // 2, body, 0)

    pl.when(cid == 0)(lambda: run(CPW0))
    pl.when(cid != 0)(lambda: run(CPW1))
    pltpu.make_async_copy(h_hbm.at[s0], r0, g0).wait()
    pltpu.make_async_copy(h_hbm.at[s1], r1, g1).wait()
    plsc.subcore_barrier()
    pltpu.sync_copy(acc_sh.at[pl.ds(sid * RPT, RPT)],
                    out_hbm.at[cid, pl.ds(sid * RPT, RPT)])
    pltpu.sync_copy(cd, cd_hbm.at[wid])


def _scale_body(cnt_ref, feat_ref, h_ref):
    deg = jnp.maximum(jnp.sum(cnt_ref[...], axis=1, keepdims=True), 1.0)
    h_ref[...] = feat_ref[...] * lax.rsqrt(deg)


_scale = pl.pallas_call(
    _scale_body,
    grid=(N_PAD // 1024,),
    in_specs=[pl.BlockSpec((1024, NW), lambda i: (i, 0)),
              pl.BlockSpec((1024, D), lambda i: (i, 0))],
    out_specs=pl.BlockSpec((1024, D), lambda i: (i, 0)),
    out_shape=jax.ShapeDtypeStruct((N_PAD, D), _F32),
)


def _final_body(a0_ref, a1_ref, cnt_ref, w_ref, b_ref, o_ref):
    a = a0_ref[...] + a1_ref[...]
    r = jnp.dot(a, w_ref[...], preferred_element_type=_F32)
    nd = lax.rsqrt(jnp.maximum(jnp.sum(cnt_ref[...], axis=1, keepdims=True), 1.0))
    o_ref[...] = r * nd + b_ref[...]


_final = pl.pallas_call(
    _final_body,
    grid=(N_PAD // 1024,),
    in_specs=[pl.BlockSpec((1024, D), lambda i: (i, 0)),
              pl.BlockSpec((1024, D), lambda i: (i, 0)),
              pl.BlockSpec((1024, NW), lambda i: (i, 0)),
              pl.BlockSpec((D, D), lambda i: (0, 0)),
              pl.BlockSpec((1, D), lambda i: (0, 0))],
    out_specs=pl.BlockSpec((1024, D), lambda i: (i, 0)),
    out_shape=jax.ShapeDtypeStruct((N_PAD, D), _F32),
)


def kernel(feat, edge_index, W, b):
    src = edge_index[0]
    dst = edge_index[1]
    pad = jnp.full((E_PAD - N_EDGES,), N_NODES, jnp.int32)
    src_p = jnp.concatenate([src, pad])
    dst_p = jnp.concatenate([dst, pad])
    feat_p = jnp.pad(feat, ((0, N_PAD - N_NODES), (0, 0)))

    cnt_s = _deg_kernel(src_p).T
    h = _scale(cnt_s, feat_p)

    zrows = jnp.zeros((RPT, D), _F32)
    acc, cd_part = _agg_kernel(h, src_p, dst_p, zrows)

    out = _final(acc[0], acc[1], cd_part.T, W, b.reshape(1, D))
    return out[:N_NODES]
